# baseline (speedup 1.0000x reference)
import jax
import jax.numpy as jnp
from jax.experimental import pallas as pl
from jax.experimental.pallas import tpu as pltpu

_TOPK = 256
_CHUNK = 1024


def _topk_mask_kernel(x_ref, o_ref, m_ref):
    x = x_ref[...]
    R, C = x.shape
    nchunk = C // 128
    u = jax.lax.bitcast_convert_type(x, jnp.uint32)
    s = u >> jnp.uint32(31)
    flip = jnp.where(s == 0, jnp.uint32(0x80000000), jnp.uint32(0xFFFFFFFF))
    m = u ^ flip
    m_ref[...] = m

    def count_ge(cand):
        candb = cand[:, None]
        acc = jnp.zeros((R, 128), jnp.float32)
        for c in range(C // _CHUNK):
            blk = m_ref[:, c * _CHUNK:(c + 1) * _CHUNK]
            ind = jnp.where(blk >= candb, jnp.float32(1.0),
                            jnp.float32(0.0))
            acc = acc + ind.reshape(R, _CHUNK // 128, 128).sum(axis=1)
        return jnp.sum(acc, axis=1)

    def body(i, carry):
        prefix, sep, septhr = carry
        b = jnp.uint32(31) - i.astype(jnp.uint32)
        cand = prefix | jnp.left_shift(jnp.uint32(1), b)
        cnt = count_ge(cand)
        newly = (sep == 0) & (cnt == jnp.float32(_TOPK))
        septhr = jnp.where(newly, cand, septhr)
        sep = jnp.where(newly, jnp.int32(1), sep)
        prefix = jnp.where(cnt >= jnp.float32(_TOPK), cand, prefix)
        return prefix, sep, septhr

    carry0 = (jnp.zeros((R,), jnp.uint32),
              jnp.zeros((R,), jnp.int32),
              jnp.zeros((R,), jnp.uint32))
    prefix, sep, septhr = jax.lax.fori_loop(0, 32, body, carry0)

    def fast_path(_):
        return jnp.where(m >= septhr[:, None], x, jnp.float32(0.0))

    def tie_path(_):
        gt = m > prefix[:, None]
        eq_f = (m == prefix[:, None]).astype(jnp.float32)
        need = (jnp.float32(_TOPK)
                - jnp.sum(gt.astype(jnp.float32), axis=1))

        e3 = eq_f.reshape(R * nchunk, 128)
        tri128 = (jax.lax.broadcasted_iota(jnp.int32, (128, 128), 0)
                  <= jax.lax.broadcasted_iota(jnp.int32, (128, 128), 1)
                  ).astype(jnp.float32)
        pref_in = jnp.dot(e3, tri128,
                          preferred_element_type=jnp.float32)
        pref_in = pref_in.reshape(R, nchunk, 128)
        chunk_tot = eq_f.reshape(R, nchunk, 128).sum(axis=2)
        trin = (jax.lax.broadcasted_iota(jnp.int32, (nchunk, nchunk), 0)
                < jax.lax.broadcasted_iota(jnp.int32, (nchunk, nchunk), 1)
                ).astype(jnp.float32)
        chunk_excl = jnp.dot(chunk_tot, trin,
                             preferred_element_type=jnp.float32)
        rank = (pref_in + chunk_excl[:, :, None]).reshape(R, C)
        keep_eq = (eq_f > 0) & (rank <= need[:, None])
        out_exact = jnp.where(gt | keep_eq, x, jnp.float32(0.0))
        out_sep = jnp.where(m >= septhr[:, None], x, jnp.float32(0.0))
        return jnp.where((sep > 0)[:, None], out_sep, out_exact)

    o_ref[...] = jax.lax.cond(jnp.min(sep) > 0, fast_path, tie_path,
                              operand=None)


@jax.jit
def kernel(x):
    R, C = x.shape
    BR = 16
    return pl.pallas_call(
        _topk_mask_kernel,
        grid=(R // BR,),
        in_specs=[pl.BlockSpec((BR, C), lambda i: (i, 0))],
        out_specs=pl.BlockSpec((BR, C), lambda i: (i, 0)),
        out_shape=jax.ShapeDtypeStruct((R, C), x.dtype),
        scratch_shapes=[pltpu.VMEM((BR, C), jnp.uint32)],
    )(x)


# speedup vs baseline: 2.2328x; 2.2328x over previous
import jax
import jax.numpy as jnp
from jax.experimental import pallas as pl

_TOPK = 256


def _topk_mask_kernel(x_ref, o_ref):
    x = x_ref[...]
    R, C = x.shape
    u = jax.lax.bitcast_convert_type(x, jnp.uint32)
    s = u >> jnp.uint32(31)
    flip = jnp.where(s == 0, jnp.uint32(0x80000000), jnp.uint32(0xFFFFFFFF))
    m = u ^ flip

    def body(i, prefix):
        b = (jnp.uint32(31) - i.astype(jnp.uint32))
        cand = prefix | jnp.left_shift(jnp.uint32(1), b)
        cnt = jnp.sum((m >= cand[:, None]).astype(jnp.int32), axis=1)
        return jnp.where(cnt >= _TOPK, cand, prefix)

    prefix0 = jnp.zeros((R,), jnp.uint32)
    thr = jax.lax.fori_loop(0, 32, body, prefix0)

    gt = m > thr[:, None]
    eq_f = (m == thr[:, None]).astype(jnp.float32)
    need = (jnp.float32(_TOPK)
            - jnp.sum(gt.astype(jnp.float32), axis=1))

    nchunk = C // 128
    e3 = eq_f.reshape(R * nchunk, 128)
    tri128 = (jax.lax.broadcasted_iota(jnp.int32, (128, 128), 0)
              <= jax.lax.broadcasted_iota(jnp.int32, (128, 128), 1)
              ).astype(jnp.float32)
    pref_in = jnp.dot(e3, tri128,
                      preferred_element_type=jnp.float32)
    pref_in = pref_in.reshape(R, nchunk, 128)
    chunk_tot = eq_f.reshape(R, nchunk, 128).sum(axis=2)
    trin = (jax.lax.broadcasted_iota(jnp.int32, (nchunk, nchunk), 0)
            < jax.lax.broadcasted_iota(jnp.int32, (nchunk, nchunk), 1)
            ).astype(jnp.float32)
    chunk_excl = jnp.dot(chunk_tot, trin,
                         preferred_element_type=jnp.float32)
    rank = (pref_in + chunk_excl[:, :, None]).reshape(R, C)
    keep_eq = (eq_f > 0) & (rank <= need[:, None])
    o_ref[...] = jnp.where(gt | keep_eq, x, jnp.float32(0.0))


@jax.jit
def kernel(x):
    R, C = x.shape
    BR = 16
    return pl.pallas_call(
        _topk_mask_kernel,
        grid=(R // BR,),
        in_specs=[pl.BlockSpec((BR, C), lambda i: (i, 0))],
        out_specs=pl.BlockSpec((BR, C), lambda i: (i, 0)),
        out_shape=jax.ShapeDtypeStruct((R, C), x.dtype),
    )(x)


# speedup vs baseline: 2.2332x; 1.0002x over previous
import jax
import jax.numpy as jnp
from jax.experimental import pallas as pl

_TOPK = 256


def _topk_mask_kernel(x_ref, o_ref):
    x = x_ref[...]
    R, C = x.shape
    u = jax.lax.bitcast_convert_type(x, jnp.uint32)
    s = u >> jnp.uint32(31)
    flip = jnp.where(s == 0, jnp.uint32(0x80000000), jnp.uint32(0xFFFFFFFF))
    m = u ^ flip

    def body(i, prefix):
        b = (jnp.uint32(31) - i.astype(jnp.uint32))
        cand = prefix | jnp.left_shift(jnp.uint32(1), b)
        cnt = jnp.sum((m >= cand).astype(jnp.int32), axis=1,
                      keepdims=True)
        return jnp.where(cnt >= _TOPK, cand, prefix)

    prefix0 = jnp.zeros((R, 1), jnp.uint32)
    thr = jax.lax.fori_loop(0, 32, body, prefix0)

    gt = m > thr
    eq_f = (m == thr).astype(jnp.float32)
    need = (jnp.float32(_TOPK)
            - jnp.sum(gt.astype(jnp.float32), axis=1,
                      keepdims=True))

    nchunk = C // 128
    e3 = eq_f.reshape(R * nchunk, 128)
    tri128 = (jax.lax.broadcasted_iota(jnp.int32, (128, 128), 0)
              <= jax.lax.broadcasted_iota(jnp.int32, (128, 128), 1)
              ).astype(jnp.float32)
    pref_in = jnp.dot(e3, tri128,
                      preferred_element_type=jnp.float32)
    pref_in = pref_in.reshape(R, nchunk, 128)
    chunk_tot = eq_f.reshape(R, nchunk, 128).sum(axis=2)
    trin = (jax.lax.broadcasted_iota(jnp.int32, (nchunk, nchunk), 0)
            < jax.lax.broadcasted_iota(jnp.int32, (nchunk, nchunk), 1)
            ).astype(jnp.float32)
    chunk_excl = jnp.dot(chunk_tot, trin,
                         preferred_element_type=jnp.float32)
    rank = (pref_in + chunk_excl[:, :, None]).reshape(R, C)
    keep_eq = (eq_f > 0) & (rank <= need)
    o_ref[...] = jnp.where(gt | keep_eq, x, jnp.float32(0.0))


@jax.jit
def kernel(x):
    R, C = x.shape
    BR = 16
    return pl.pallas_call(
        _topk_mask_kernel,
        grid=(R // BR,),
        in_specs=[pl.BlockSpec((BR, C), lambda i: (i, 0))],
        out_specs=pl.BlockSpec((BR, C), lambda i: (i, 0)),
        out_shape=jax.ShapeDtypeStruct((R, C), x.dtype),
    )(x)


# speedup vs baseline: 2.4696x; 1.1058x over previous
import jax
import jax.numpy as jnp
from jax.experimental import pallas as pl

_TOPK = 256


def _topk_mask_kernel(x_ref, o_ref):
    x = x_ref[...]
    R, C = x.shape
    u = jax.lax.bitcast_convert_type(x, jnp.uint32)
    s = u >> jnp.uint32(31)
    flip = jnp.where(s == 0, jnp.uint32(0x80000000), jnp.uint32(0xFFFFFFFF))
    m = u ^ flip

    def body(i, carry):
        prefix, exact = carry
        b = (jnp.uint32(31) - i.astype(jnp.uint32))
        cand = prefix | jnp.left_shift(jnp.uint32(1), b)
        cnt = jnp.sum((m >= cand).astype(jnp.int32), axis=1,
                      keepdims=True)
        exact = exact | (cnt == _TOPK)
        return jnp.where(cnt >= _TOPK, cand, prefix), exact

    carry0 = (jnp.zeros((R, 1), jnp.uint32), jnp.zeros((R, 1), jnp.int32))
    thr, exact = jax.lax.fori_loop(0, 32, body, carry0)

    def fast_path(_):
        return jnp.where(m >= thr, x, jnp.float32(0.0))

    def tie_path(_):
        gt = m > thr
        eq_f = (m == thr).astype(jnp.float32)
        need = (jnp.float32(_TOPK)
                - jnp.sum(gt.astype(jnp.float32), axis=1,
                          keepdims=True))

        nchunk = C // 128
        e3 = eq_f.reshape(R * nchunk, 128)
        tri128 = (jax.lax.broadcasted_iota(jnp.int32, (128, 128), 0)
                  <= jax.lax.broadcasted_iota(jnp.int32, (128, 128), 1)
                  ).astype(jnp.float32)
        pref_in = jnp.dot(e3, tri128,
                          preferred_element_type=jnp.float32)
        pref_in = pref_in.reshape(R, nchunk, 128)
        chunk_tot = eq_f.reshape(R, nchunk, 128).sum(axis=2)
        trin = (jax.lax.broadcasted_iota(jnp.int32, (nchunk, nchunk), 0)
                < jax.lax.broadcasted_iota(jnp.int32, (nchunk, nchunk), 1)
                ).astype(jnp.float32)
        chunk_excl = jnp.dot(chunk_tot, trin,
                             preferred_element_type=jnp.float32)
        rank = (pref_in + chunk_excl[:, :, None]).reshape(R, C)
        keep_eq = (eq_f > 0) & (rank <= need)
        return jnp.where(gt | keep_eq, x, jnp.float32(0.0))

    o_ref[...] = jax.lax.cond(jnp.min(exact) > 0, fast_path, tie_path,
                              operand=None)


@jax.jit
def kernel(x):
    R, C = x.shape
    BR = 16
    return pl.pallas_call(
        _topk_mask_kernel,
        grid=(R // BR,),
        in_specs=[pl.BlockSpec((BR, C), lambda i: (i, 0))],
        out_specs=pl.BlockSpec((BR, C), lambda i: (i, 0)),
        out_shape=jax.ShapeDtypeStruct((R, C), x.dtype),
    )(x)


# speedup vs baseline: 2.7400x; 1.1095x over previous
import jax
import jax.numpy as jnp
from jax.experimental import pallas as pl

_TOPK = 256


def _topk_mask_kernel(x_ref, o_ref):
    x = x_ref[...]
    R, C = x.shape
    u = jax.lax.bitcast_convert_type(x, jnp.uint32)
    s = u >> jnp.uint32(31)
    flip = jnp.where(s == 0, jnp.uint32(0x80000000), jnp.uint32(0xFFFFFFFF))
    m = u ^ flip

    def body(i, carry):
        prefix, exact = carry
        b = (jnp.uint32(31) - i.astype(jnp.uint32))
        cand = prefix | jnp.left_shift(jnp.uint32(1), b)
        cnt = jnp.sum((m >= cand).astype(jnp.int32), axis=1,
                      keepdims=True)
        exact = exact | (cnt == _TOPK)
        return jnp.where(cnt >= _TOPK, cand, prefix), exact

    carry0 = (jnp.zeros((R, 1), jnp.uint32), jnp.zeros((R, 1), jnp.int32))
    thr, exact = jax.lax.fori_loop(0, 32, body, carry0)

    def fast_path(_):
        return jnp.where(m >= thr, x, jnp.float32(0.0))

    def tie_path(_):
        gt = m > thr
        eq_f = (m == thr).astype(jnp.float32)
        need = (jnp.float32(_TOPK)
                - jnp.sum(gt.astype(jnp.float32), axis=1,
                          keepdims=True))

        nchunk = C // 128
        e3 = eq_f.reshape(R * nchunk, 128)
        tri128 = (jax.lax.broadcasted_iota(jnp.int32, (128, 128), 0)
                  <= jax.lax.broadcasted_iota(jnp.int32, (128, 128), 1)
                  ).astype(jnp.float32)
        pref_in = jnp.dot(e3, tri128,
                          preferred_element_type=jnp.float32)
        pref_in = pref_in.reshape(R, nchunk, 128)
        chunk_tot = eq_f.reshape(R, nchunk, 128).sum(axis=2)
        trin = (jax.lax.broadcasted_iota(jnp.int32, (nchunk, nchunk), 0)
                < jax.lax.broadcasted_iota(jnp.int32, (nchunk, nchunk), 1)
                ).astype(jnp.float32)
        chunk_excl = jnp.dot(chunk_tot, trin,
                             preferred_element_type=jnp.float32)
        rank = (pref_in + chunk_excl[:, :, None]).reshape(R, C)
        keep_eq = (eq_f > 0) & (rank <= need)
        return jnp.where(gt | keep_eq, x, jnp.float32(0.0))

    o_ref[...] = jax.lax.cond(jnp.min(exact) > 0, fast_path, tie_path,
                              operand=None)


@jax.jit
def kernel(x):
    R, C = x.shape
    BR = 32
    return pl.pallas_call(
        _topk_mask_kernel,
        grid=(R // BR,),
        in_specs=[pl.BlockSpec((BR, C), lambda i: (i, 0))],
        out_specs=pl.BlockSpec((BR, C), lambda i: (i, 0)),
        out_shape=jax.ShapeDtypeStruct((R, C), x.dtype),
    )(x)
